# baseline (speedup 1.0000x reference)
import functools
import math

import jax
import jax.numpy as jnp
from jax import lax
from jax.experimental import pallas as pl
from jax.experimental.pallas import tpu as pltpu
from jax.experimental.pallas import tpu_sc as plsc

N = 16384
EMB = 512
K = EMB // 4
TAB = 2048
NC, NS, L = 2, 16, 16
NW = NC * NS
RPW = N // NW
C = 32
NCH = RPW // C


CB = C * EMB


def _sc_body(x_hbm, y_hbm, gidx_hbm, src_hbm, out_hbm,
             idx_all, pe_a, pe_b, out_a, out_b,
             gs_a, gs_b, ss_a, ss_b, os_a, os_b):
    wid = lax.axis_index("s") * NC + lax.axis_index("c")
    pltpu.sync_copy(gidx_hbm.at[pl.ds(4 * wid, 4)], idx_all)

    tabs = (x_hbm, y_hbm, x_hbm, y_hbm)
    pe = (pe_a, pe_b)
    outb = (out_a, out_b)
    gsem = (gs_a, gs_b)
    ssem = (ss_a, ss_b)
    osem = (os_a, os_b)
    descs = {}

    def start(ch):
        b = ch & 1
        g = wid * NCH + ch
        t, mo = ch // 4, C * (ch % 4)
        descs["g", b] = [
            pltpu.async_copy(
                tabs[j].at[idx_all.at[t, j, pl.ds(mo, C)]],
                pe[b].at[pl.ds(j * C, C)], gsem[b])
            for j in range(4)
        ]
        descs["s", b] = pltpu.async_copy(src_hbm.at[g], outb[b], ssem[b])

    start(0)
    for ch in range(NCH):
        b = ch & 1
        if ch + 1 < NCH:
            if ch >= 1:
                descs["o", 1 - b].wait()
            start(ch + 1)
        for d in descs["g", b]:
            d.wait()
        descs["s", b].wait()

        _pe, _out = pe[b], outb[b]

        @plsc.parallel_loop(0, C)
        def _rows(c, _pe=_pe, _out=_out):
            rbase = (c // 8) * 4096 + (c % 8) * 128
            for j in range(4):
                for u in range(K // L):
                    v = _pe[C * j + c, pl.ds(L * u, L)]
                    pat = (4 * lax.iota(jnp.int32, L)
                           + (1024 * (u // 2) + 64 * (u % 2) + j))
                    plsc.addupdate_scatter(_out, [rbase + pat], v)

        g = wid * NCH + ch
        descs["o", b] = pltpu.async_copy(outb[b], out_hbm.at[g], osem[b])

    descs["o", 0].wait()
    descs["o", 1].wait()


@jax.jit
def _source_pe_sc(x_pe, y_pe, gidx3, src_tiles):
    mesh = plsc.VectorSubcoreMesh(core_axis_name="c", subcore_axis_name="s")
    run = pl.kernel(
        _sc_body,
        out_type=jax.ShapeDtypeStruct((NW * NCH, CB), jnp.float32),
        mesh=mesh,
        scratch_types=[
            pltpu.VMEM((4, 4, 4 * C), jnp.int32),
            pltpu.VMEM((4 * C, K), jnp.float32),
            pltpu.VMEM((4 * C, K), jnp.float32),
            pltpu.VMEM((CB,), jnp.float32),
            pltpu.VMEM((CB,), jnp.float32),
            pltpu.SemaphoreType.DMA,
            pltpu.SemaphoreType.DMA,
            pltpu.SemaphoreType.DMA,
            pltpu.SemaphoreType.DMA,
            pltpu.SemaphoreType.DMA,
            pltpu.SemaphoreType.DMA,
        ],
        compiler_params=pltpu.CompilerParams(
            use_tc_tiling_on_sc=False, needs_layout_passes=False
        ),
    )
    return run(x_pe, y_pe, gidx3, src_tiles)


def kernel(src_embedding, src_boxes, x_pe, y_pe):
    gidx3 = src_boxes.reshape(N // 128, 128, 4).transpose(0, 2, 1)
    src_tiles = (src_embedding.reshape(N // 8, 8, EMB // 128, 128)
                 .transpose(0, 2, 1, 3).reshape(NW * NCH, CB))
    out_tiles = _source_pe_sc(x_pe, y_pe, gidx3, src_tiles)
    return (out_tiles.reshape(N // 8, EMB // 128, 8, 128)
            .transpose(0, 2, 1, 3).reshape(N, EMB))


# speedup vs baseline: 1.0028x; 1.0028x over previous
import functools
import math

import jax
import jax.numpy as jnp
from jax import lax
from jax.experimental import pallas as pl
from jax.experimental.pallas import tpu as pltpu
from jax.experimental.pallas import tpu_sc as plsc

N = 16384
EMB = 512
K = EMB // 4
TAB = 2048
NC, NS, L = 2, 16, 16
NW = NC * NS
RPW = N // NW
C = 32
NCH = RPW // C


CB = C * EMB


def _sc_body(x_hbm, y_hbm, gidx_hbm, src_hbm, out_hbm,
             idx_all, pe_a, pe_b, out_a, out_b, shx, shy,
             gs_a, gs_b, ss_a, ss_b, os_a, os_b):
    wid = lax.axis_index("s") * NC + lax.axis_index("c")
    @pl.when(lax.axis_index("s") == 0)
    def _stage():
        pltpu.sync_copy(x_hbm, shx)
        pltpu.sync_copy(y_hbm, shy)

    pltpu.sync_copy(gidx_hbm.at[pl.ds(4 * wid, 4)], idx_all)
    plsc.subcore_barrier()

    tabs = (shx, shy, shx, shy)
    pe = (pe_a, pe_b)
    outb = (out_a, out_b)
    gsem = (gs_a, gs_b)
    ssem = (ss_a, ss_b)
    osem = (os_a, os_b)
    descs = {}

    def start(ch):
        b = ch & 1
        g = wid * NCH + ch
        t, mo = ch // 4, C * (ch % 4)
        descs["g", b] = [
            pltpu.async_copy(
                tabs[j].at[idx_all.at[t, j, pl.ds(mo, C)]],
                pe[b].at[pl.ds(j * C, C)], gsem[b])
            for j in range(4)
        ]
        descs["s", b] = pltpu.async_copy(src_hbm.at[g], outb[b], ssem[b])

    start(0)
    for ch in range(NCH):
        b = ch & 1
        if ch + 1 < NCH:
            if ch >= 1:
                descs["o", 1 - b].wait()
            start(ch + 1)
        for d in descs["g", b]:
            d.wait()
        descs["s", b].wait()

        _pe, _out = pe[b], outb[b]

        @plsc.parallel_loop(0, C)
        def _rows(c, _pe=_pe, _out=_out):
            rbase = (c // 8) * 4096 + (c % 8) * 128
            for j in range(4):
                for u in range(K // L):
                    v = _pe[C * j + c, pl.ds(L * u, L)]
                    pat = (4 * lax.iota(jnp.int32, L)
                           + (1024 * (u // 2) + 64 * (u % 2) + j))
                    plsc.addupdate_scatter(_out, [rbase + pat], v)

        g = wid * NCH + ch
        descs["o", b] = pltpu.async_copy(outb[b], out_hbm.at[g], osem[b])

    descs["o", 0].wait()
    descs["o", 1].wait()


@jax.jit
def _source_pe_sc(x_pe, y_pe, gidx3, src_tiles):
    mesh = plsc.VectorSubcoreMesh(core_axis_name="c", subcore_axis_name="s")
    run = pl.kernel(
        _sc_body,
        out_type=jax.ShapeDtypeStruct((NW * NCH, CB), jnp.float32),
        mesh=mesh,
        scratch_types=[
            pltpu.VMEM((4, 4, 4 * C), jnp.int32),
            pltpu.VMEM((4 * C, K), jnp.float32),
            pltpu.VMEM((4 * C, K), jnp.float32),
            pltpu.VMEM((CB,), jnp.float32),
            pltpu.VMEM((CB,), jnp.float32),
            pltpu.VMEM_SHARED((1024, K), jnp.float32),
            pltpu.VMEM_SHARED((1024, K), jnp.float32),
            pltpu.SemaphoreType.DMA,
            pltpu.SemaphoreType.DMA,
            pltpu.SemaphoreType.DMA,
            pltpu.SemaphoreType.DMA,
            pltpu.SemaphoreType.DMA,
            pltpu.SemaphoreType.DMA,
        ],
        compiler_params=pltpu.CompilerParams(
            use_tc_tiling_on_sc=False, needs_layout_passes=False
        ),
    )
    return run(x_pe, y_pe, gidx3, src_tiles)


def kernel(src_embedding, src_boxes, x_pe, y_pe):
    gidx3 = src_boxes.reshape(N // 128, 128, 4).transpose(0, 2, 1)
    src_tiles = (src_embedding.reshape(N // 8, 8, EMB // 128, 128)
                 .transpose(0, 2, 1, 3).reshape(NW * NCH, CB))
    out_tiles = _source_pe_sc(x_pe, y_pe, gidx3, src_tiles)
    return (out_tiles.reshape(N // 8, EMB // 128, 8, 128)
            .transpose(0, 2, 1, 3).reshape(N, EMB))


# speedup vs baseline: 1.0141x; 1.0113x over previous
import functools
import math

import jax
import jax.numpy as jnp
from jax import lax
from jax.experimental import pallas as pl
from jax.experimental.pallas import tpu as pltpu
from jax.experimental.pallas import tpu_sc as plsc

N = 16384
EMB = 512
K = EMB // 4
TAB = 2048
NC, NS, L = 2, 16, 16
NW = NC * NS
RPW = N // NW
C = 32
NCH = RPW // C


CB = C * EMB


def _sc_body(x_hbm, y_hbm, gidx_hbm, src_hbm, out_hbm,
             idx_all, pe_a, pe_b, pe_c, out_a, out_b, out_c,
             gs_a, gs_b, gs_c, ss_a, ss_b, ss_c, os_a, os_b, os_c):
    wid = lax.axis_index("s") * NC + lax.axis_index("c")
    pltpu.sync_copy(gidx_hbm.at[pl.ds(4 * wid, 4)], idx_all)

    tabs = (x_hbm, y_hbm, x_hbm, y_hbm)
    pe = (pe_a, pe_b, pe_c)
    outb = (out_a, out_b, out_c)
    gsem = (gs_a, gs_b, gs_c)
    ssem = (ss_a, ss_b, ss_c)
    osem = (os_a, os_b, os_c)
    NB = 3
    descs = {}

    def start(ch):
        b = ch % NB
        g = wid * NCH + ch
        t, mo = ch // 4, C * (ch % 4)
        descs["g", b] = [
            pltpu.async_copy(
                tabs[j].at[idx_all.at[t, j, pl.ds(mo, C)]],
                pe[b].at[pl.ds(j * C, C)], gsem[b])
            for j in range(4)
        ]
        descs["s", b] = pltpu.async_copy(src_hbm.at[g], outb[b], ssem[b])

    start(0)
    start(1)
    for ch in range(NCH):
        b = ch % NB
        nxt = ch + 2
        if nxt < NCH:
            nb = nxt % NB
            if ("o", nb) in descs:
                descs["o", nb].wait()
            start(nxt)
        for d in descs["g", b]:
            d.wait()
        descs["s", b].wait()

        _pe, _out = pe[b], outb[b]

        @plsc.parallel_loop(0, C)
        def _rows(c, _pe=_pe, _out=_out):
            rbase = (c // 8) * 4096 + (c % 8) * 128
            for j in range(4):
                for u in range(K // L):
                    v = _pe[C * j + c, pl.ds(L * u, L)]
                    pat = (4 * lax.iota(jnp.int32, L)
                           + (1024 * (u // 2) + 64 * (u % 2) + j))
                    plsc.addupdate_scatter(_out, [rbase + pat], v)

        g = wid * NCH + ch
        descs["o", b] = pltpu.async_copy(outb[b], out_hbm.at[g], osem[b])

    descs["o", 0].wait()
    descs["o", 1].wait()
    descs["o", 2].wait()


@jax.jit
def _source_pe_sc(x_pe, y_pe, gidx3, src_tiles):
    mesh = plsc.VectorSubcoreMesh(core_axis_name="c", subcore_axis_name="s")
    run = pl.kernel(
        _sc_body,
        out_type=jax.ShapeDtypeStruct((NW * NCH, CB), jnp.float32),
        mesh=mesh,
        scratch_types=[
            pltpu.VMEM((4, 4, 4 * C), jnp.int32),
            pltpu.VMEM((4 * C, K), jnp.float32),
            pltpu.VMEM((4 * C, K), jnp.float32),
            pltpu.VMEM((4 * C, K), jnp.float32),
            pltpu.VMEM((CB,), jnp.float32),
            pltpu.VMEM((CB,), jnp.float32),
            pltpu.VMEM((CB,), jnp.float32),
        ] + [pltpu.SemaphoreType.DMA] * 9,
        compiler_params=pltpu.CompilerParams(
            use_tc_tiling_on_sc=False, needs_layout_passes=False
        ),
    )
    return run(x_pe, y_pe, gidx3, src_tiles)


def kernel(src_embedding, src_boxes, x_pe, y_pe):
    gidx3 = src_boxes.reshape(N // 128, 128, 4).transpose(0, 2, 1)
    src_tiles = (src_embedding.reshape(N // 8, 8, EMB // 128, 128)
                 .transpose(0, 2, 1, 3).reshape(NW * NCH, CB))
    out_tiles = _source_pe_sc(x_pe, y_pe, gidx3, src_tiles)
    return (out_tiles.reshape(N // 8, EMB // 128, 8, 128)
            .transpose(0, 2, 1, 3).reshape(N, EMB))
